# baseline (speedup 1.0000x reference)
import functools

import jax
import jax.numpy as jnp
from jax import lax
from jax.experimental import pallas as pl
from jax.experimental.pallas import tpu as pltpu
from jax.experimental.pallas import tpu_sc as plsc

D = 768
H = 12
DH = 64
T = 8
NPF = 196
S = T * NPF
E = 8
K = 2
DFF = 3072
QB = 4
EPS = 1e-5

NT = 2 * S
BLK = 256
NB = 20
PADDED = NB * BLK
CW = 112


def _embed_body(p_ref, w_ref, b_ref, pos_ref, temp_ref, o_ref):
    mm = jnp.dot(p_ref[...], w_ref[...], preferred_element_type=jnp.float32)
    add = pos_ref[...][None, :, :] + temp_ref[...][:, None, :]
    o_ref[...] = mm + b_ref[...][None, :] + add.reshape(S, D)


def _embed(patches, w, b, pos, temp):
    return pl.pallas_call(
        _embed_body,
        out_shape=jax.ShapeDtypeStruct((S, D), jnp.float32),
    )(patches, w, b, pos, temp)


def _ln_body(x_ref, g_ref, b_ref, o_ref):
    x = x_ref[...]
    m = jnp.mean(x, axis=-1, keepdims=True)
    v = jnp.mean((x - m) ** 2, axis=-1, keepdims=True)
    o_ref[...] = (x - m) * jax.lax.rsqrt(v + EPS) * g_ref[...][None, :] + b_ref[...][None, :]


def _layernorm(x, g, b):
    return pl.pallas_call(
        _ln_body,
        out_shape=jax.ShapeDtypeStruct(x.shape, jnp.float32),
    )(x, g, b)


def _qkv_body(h_ref, wq_ref, wk_ref, wv_ref, bq_ref, bk_ref, bv_ref,
              q_ref, k_ref, v_ref):
    h = h_ref[...]
    q_ref[0] = jnp.dot(h, wq_ref[0], preferred_element_type=jnp.float32) + bq_ref[0]
    k_ref[0] = jnp.dot(h, wk_ref[0], preferred_element_type=jnp.float32) + bk_ref[0]
    v_ref[0] = jnp.dot(h, wv_ref[0], preferred_element_type=jnp.float32) + bv_ref[0]


def _qkv(hln, wq, wk, wv, bq, bk, bv):
    spec_w = pl.BlockSpec((1, D, DH), lambda h: (h, 0, 0))
    spec_b = pl.BlockSpec((1, 1, DH), lambda h: (h, 0, 0))
    spec_o = pl.BlockSpec((1, S, DH), lambda h: (h, 0, 0))
    return pl.pallas_call(
        _qkv_body,
        grid=(H,),
        in_specs=[pl.BlockSpec((S, D), lambda h: (0, 0)),
                  spec_w, spec_w, spec_w, spec_b, spec_b, spec_b],
        out_specs=[spec_o, spec_o, spec_o],
        out_shape=[jax.ShapeDtypeStruct((H, S, DH), jnp.float32)] * 3,
    )(hln, wq, wk, wv, bq, bk, bv)


def _attn_body(q_ref, k_ref, v_ref, tb_ref, o_ref):
    qb = pl.program_id(1)
    q = q_ref[0]
    k = k_ref[0]
    v = v_ref[0]
    tb = tb_ref[0]
    logits = jax.lax.dot_general(
        q, k, (((1,), (1,)), ((), ())),
        preferred_element_type=jnp.float32) * (1.0 / 8.0)
    fr_iota = jax.lax.broadcasted_iota(jnp.int32, (T, T), 0)
    selA = (fr_iota == 2 * qb).astype(jnp.float32)
    selB = (fr_iota == 2 * qb + 1).astype(jnp.float32)
    tbA = jnp.sum(tb * selA, axis=0, keepdims=True)
    tbB = jnp.sum(tb * selB, axis=0, keepdims=True)
    colf = jax.lax.broadcasted_iota(jnp.int32, (1, S), 1) // NPF
    rowA = jnp.zeros((1, S), jnp.float32)
    rowB = jnp.zeros((1, S), jnp.float32)
    for f in range(T):
        m = (colf == f).astype(jnp.float32)
        rowA = rowA + m * tbA[:, f:f + 1]
        rowB = rowB + m * tbB[:, f:f + 1]
    rin = jax.lax.broadcasted_iota(jnp.int32, (2 * NPF, 1), 0)
    bias = jnp.where(rin < NPF, rowA, rowB)
    logits = logits + bias
    m = jnp.max(logits, axis=-1, keepdims=True)
    p = jnp.exp(logits - m)
    a = p / jnp.sum(p, axis=-1, keepdims=True)
    o_ref[0] = jnp.dot(a, v, preferred_element_type=jnp.float32)


def _attention(q, k, v, tbias):
    SQ = S // QB
    return pl.pallas_call(
        _attn_body,
        grid=(H, QB),
        in_specs=[pl.BlockSpec((1, SQ, DH), lambda h, qb: (h, qb, 0)),
                  pl.BlockSpec((1, S, DH), lambda h, qb: (h, 0, 0)),
                  pl.BlockSpec((1, S, DH), lambda h, qb: (h, 0, 0)),
                  pl.BlockSpec((1, T, T), lambda h, qb: (h, 0, 0))],
        out_specs=pl.BlockSpec((1, SQ, DH), lambda h, qb: (h, qb, 0)),
        out_shape=jax.ShapeDtypeStruct((H, S, DH), jnp.float32),
    )(q, k, v, tbias)


def _proj_body(o_ref, wo_ref, bo_ref, x_ref, y_ref):
    h = pl.program_id(0)

    @pl.when(h == 0)
    def _():
        y_ref[...] = x_ref[...] + bo_ref[...][None, :]

    y_ref[...] += jnp.dot(o_ref[0], wo_ref[0], preferred_element_type=jnp.float32)


def _proj_residual(o_heads, wo, bo, x):
    return pl.pallas_call(
        _proj_body,
        grid=(H,),
        in_specs=[pl.BlockSpec((1, S, DH), lambda h: (h, 0, 0)),
                  pl.BlockSpec((1, DH, D), lambda h: (h, 0, 0)),
                  pl.BlockSpec((D,), lambda h: (0,)),
                  pl.BlockSpec((S, D), lambda h: (0, 0))],
        out_specs=pl.BlockSpec((S, D), lambda h: (0, 0)),
        out_shape=jax.ShapeDtypeStruct((S, D), jnp.float32),
    )(o_heads, wo, bo, x)


def _router_body(x_ref, g_ref, b_ref, wr_ref, br_ref, ts_ref, wt_ref, h_ref, rl_ref):
    x = x_ref[...]
    m = jnp.mean(x, axis=-1, keepdims=True)
    v = jnp.mean((x - m) ** 2, axis=-1, keepdims=True)
    h = (x - m) * jax.lax.rsqrt(v + EPS) * g_ref[...][None, :] + b_ref[...][None, :]
    h_ref[...] = h
    tbias = jnp.dot(ts_ref[...], wt_ref[...], preferred_element_type=jnp.float32)
    rl_ref[...] = (jnp.dot(h, wr_ref[...], preferred_element_type=jnp.float32)
                   + br_ref[...][None, :] + tbias)


def _router(x, g, b, wr, br, text_state, wt):
    return pl.pallas_call(
        _router_body,
        out_shape=[jax.ShapeDtypeStruct((S, D), jnp.float32),
                   jax.ShapeDtypeStruct((S, E), jnp.float32)],
    )(x, g, b, wr, br, text_state, wt)


def _route_body(rl_ref, gd_ref, loads_ref, pos_ref, ebact_ref):
    r = rl_ref[...]
    iota = jax.lax.broadcasted_iota(jnp.int32, (S, E), 1)
    m1 = jnp.max(r, axis=1, keepdims=True)
    i1 = jnp.min(jnp.where(r == m1, iota, E), axis=1, keepdims=True)
    mask1b = iota == i1
    r2 = jnp.where(mask1b, -jnp.inf, r)
    m2 = jnp.max(r2, axis=1, keepdims=True)
    i2 = jnp.min(jnp.where(r2 == m2, iota, E), axis=1, keepdims=True)
    mask2b = iota == i2
    d = jnp.exp(m2 - m1)
    g1 = 1.0 / (1.0 + d)
    g2 = d / (1.0 + d)
    gd = jnp.where(mask1b, g1, 0.0) + jnp.where(mask2b, g2, 0.0)
    gd_ref[...] = gd
    loads_ref[...] = jnp.sum(gd, axis=0, keepdims=True) * (1.0 / S)

    mask1 = mask1b.astype(jnp.float32)
    mask2 = mask2b.astype(jnp.float32)
    ri = jax.lax.broadcasted_iota(jnp.int32, (S, S), 0)
    ci = jax.lax.broadcasted_iota(jnp.int32, (S, S), 1)
    tri = (ri >= ci).astype(jnp.float32)
    m12 = jnp.concatenate([mask1, mask2], axis=1)
    c12 = jnp.dot(tri, m12, preferred_element_type=jnp.float32,
                  precision=jax.lax.Precision.HIGHEST)
    c1 = c12[:, :E]
    c2 = c12[:, E:]
    cnt1 = c1[S - 1:S, :]
    cnt = cnt1 + c2[S - 1:S, :]
    nb = jnp.floor((cnt + float(BLK - 1)) * (1.0 / BLK))
    ei = jax.lax.broadcasted_iota(jnp.int32, (E, E), 0)
    ej = jax.lax.broadcasted_iota(jnp.int32, (E, E), 1)
    triE = (ei <= ej).astype(jnp.float32)
    cumnb = jnp.dot(nb, triE, preferred_element_type=jnp.float32,
                    precision=jax.lax.Precision.HIGHEST)
    seg = (cumnb - nb) * float(BLK)
    rank1 = c1 - mask1
    rank2 = cnt1 + c2 - mask2
    pos1 = jnp.sum(mask1 * (seg + rank1), axis=1, keepdims=True)
    pos2 = jnp.sum(mask2 * (seg + rank2), axis=1, keepdims=True)
    pos_ref[...] = jnp.concatenate([pos1, pos2], axis=1).astype(jnp.int32)

    bif = jax.lax.broadcasted_iota(jnp.int32, (2 * E * K, 1), 0).astype(jnp.float32)
    ebcol = jnp.sum((bif >= cumnb).astype(jnp.float32), axis=1, keepdims=True)
    ebcol = jnp.minimum(ebcol, float(E - 1))
    actcol = (bif < cumnb[:, E - 1:E]).astype(jnp.float32)
    ebact_ref[...] = jnp.concatenate([ebcol, actcol], axis=1).astype(jnp.int32)


def _route_tc(rl):
    return pl.pallas_call(
        _route_body,
        out_shape=[jax.ShapeDtypeStruct((S, E), jnp.float32),
                   jax.ShapeDtypeStruct((1, E), jnp.float32),
                   jax.ShapeDtypeStruct((S, 2), jnp.int32),
                   jax.ShapeDtypeStruct((32, 2), jnp.int32)],
    )(rl)


_SC_MESH = dict(core_axis_name="c", subcore_axis_name="s")


def _sc_scatter_body(h2_hbm, pos_hbm, out_hbm, idxv, rows, sem):
    c = lax.axis_index("c")
    s = lax.axis_index("s")
    w = s * 2 + c

    @pl.when(w < 28)
    def _():
        abase = w * CW
        tbase = abase - jnp.where(abase >= S, S, 0)
        pltpu.sync_copy(pos_hbm.at[pl.ds(abase, CW)], idxv)
        pltpu.sync_copy(h2_hbm.at[pl.ds(tbase, CW)], rows)
        pltpu.async_copy(rows, out_hbm.at[idxv], sem).wait()


def _sc_scatter(h2, pos_flat):
    f = functools.partial(
        pl.kernel,
        out_type=jax.ShapeDtypeStruct((PADDED, D), jnp.float32),
        mesh=plsc.VectorSubcoreMesh(**_SC_MESH),
        scratch_types=[
            pltpu.VMEM((CW,), jnp.int32),
            pltpu.VMEM((CW, D), jnp.float32),
            pltpu.SemaphoreType.DMA,
        ])(_sc_scatter_body)
    return f(h2, pos_flat)


def _sc_comb_body(cs_hbm, pos_hbm, out_hbm, idxv, rows, sem):
    c = lax.axis_index("c")
    s = lax.axis_index("s")
    w = s * 2 + c

    @pl.when(w < 28)
    def _():
        abase = w * CW
        pltpu.sync_copy(pos_hbm.at[pl.ds(abase, CW)], idxv)
        pltpu.async_copy(cs_hbm.at[idxv], rows, sem).wait()
        pltpu.sync_copy(rows, out_hbm.at[pl.ds(abase, CW)])


def _sc_comb(cs, pos_flat):
    f = functools.partial(
        pl.kernel,
        out_type=jax.ShapeDtypeStruct((NT, D), jnp.float32),
        mesh=plsc.VectorSubcoreMesh(**_SC_MESH),
        scratch_types=[
            pltpu.VMEM((CW,), jnp.int32),
            pltpu.VMEM((CW, D), jnp.float32),
            pltpu.SemaphoreType.DMA,
        ])(_sc_comb_body)
    return f(cs, pos_flat)


def _gmm_body(eb_ref, act_ref, rows_ref, w1_ref, bb1_ref, w2_ref, bb2_ref, o_ref):
    b = pl.program_id(0)

    @pl.when(act_ref[b] == 1)
    def _():
        hh = jax.nn.gelu(
            jnp.dot(rows_ref[...].astype(jnp.bfloat16), w1_ref[0],
                    preferred_element_type=jnp.float32) + bb1_ref[0])
        o_ref[...] = (jnp.dot(hh.astype(jnp.bfloat16), w2_ref[0],
                              preferred_element_type=jnp.float32)
                      + bb2_ref[0])


def _gmm(rows_sorted, w1, bb1, w2, bb2, eb, act):
    grid_spec = pltpu.PrefetchScalarGridSpec(
        num_scalar_prefetch=2,
        grid=(NB,),
        in_specs=[
            pl.BlockSpec((BLK, D), lambda b, eb, act: (b, 0)),
            pl.BlockSpec((1, D, DFF), lambda b, eb, act: (eb[b], 0, 0)),
            pl.BlockSpec((1, 1, DFF), lambda b, eb, act: (eb[b], 0, 0)),
            pl.BlockSpec((1, DFF, D), lambda b, eb, act: (eb[b], 0, 0)),
            pl.BlockSpec((1, 1, D), lambda b, eb, act: (eb[b], 0, 0)),
        ],
        out_specs=pl.BlockSpec((BLK, D), lambda b, eb, act: (b, 0)),
    )
    return pl.pallas_call(
        _gmm_body,
        grid_spec=grid_spec,
        out_shape=jax.ShapeDtypeStruct((PADDED, D), jnp.float32),
    )(eb, act, rows_sorted, w1, bb1, w2, bb2)


def _combine_body(x_ref, gd_ref, c1_ref, c2_ref, o_ref):
    g1 = jnp.max(gd_ref[...], axis=1, keepdims=True)
    g2 = 1.0 - g1
    o_ref[...] = x_ref[...] + g1 * c1_ref[...] + g2 * c2_ref[...]


def _combine(x, gd, c12):
    return pl.pallas_call(
        _combine_body,
        grid=(1,),
        in_specs=[pl.BlockSpec((S, D), lambda i: (0, 0)),
                  pl.BlockSpec((S, E), lambda i: (0, 0)),
                  pl.BlockSpec((S, D), lambda i: (0, 0)),
                  pl.BlockSpec((S, D), lambda i: (1, 0))],
        out_specs=pl.BlockSpec((S, D), lambda i: (0, 0)),
        out_shape=jax.ShapeDtypeStruct((S, D), jnp.float32),
    )(x, gd, c12, c12)


@jax.jit
def _run(video, text_state, params):
    P = 16
    B_, T_, C_, Hh, Ww = video.shape
    nps = Hh // P
    patches = video.reshape(B_, T_, C_, nps, P, nps, P)
    patches = patches.transpose(0, 1, 3, 5, 2, 4, 6).reshape(S, C_ * P * P)

    x = _embed(patches, params['W_patch'], params['b_patch'],
               params['pos'], params['temp'])

    loads = []
    for lp in params['layers']:
        hln = _layernorm(x, lp['g1'], lp['b1'])
        wqkv = lp['Wqkv'].reshape(D, 3, H, DH)
        wq = wqkv[:, 0].transpose(1, 0, 2)
        wk = wqkv[:, 1].transpose(1, 0, 2)
        wv = wqkv[:, 2].transpose(1, 0, 2)
        bqkv = lp['bqkv'].reshape(3, H, 1, DH)
        q, k, v = _qkv(hln, wq, wk, wv, bqkv[0], bqkv[1], bqkv[2])
        o_heads = _attention(q, k, v, lp['tbias'])
        wo = lp['Wo'].reshape(H, DH, D)
        x = _proj_residual(o_heads, wo, lp['bo'], x)

        h2, rl = _router(x, lp['g2'], lp['b2'], lp['Wr'], lp['br'],
                         text_state, lp['Wt'])
        gd, ld, posT, ebact = _route_tc(rl)
        pos_flat = posT.T.reshape(NT)
        eb = ebact[:, 0]
        act = ebact[:, 1]
        rows_sorted = _sc_scatter(h2, pos_flat)
        out_sorted = _gmm(rows_sorted,
                          lp['W1'].astype(jnp.bfloat16),
                          lp['bb1'].reshape(E, 1, DFF),
                          lp['W2'].astype(jnp.bfloat16),
                          lp['bb2'].reshape(E, 1, D), eb, act)
        c12 = _sc_comb(out_sorted, pos_flat)
        x = _combine(x, gd, c12)
        loads.append(ld[0])

    x = _layernorm(x, params['g_f'], params['b_f'])
    return x.reshape(B_, S, D), jnp.stack(loads)


def kernel(video, text_state, params):
    return _run(video, text_state, params)


# speedup vs baseline: 1.1188x; 1.1188x over previous
import functools

import jax
import jax.numpy as jnp
from jax import lax
from jax.experimental import pallas as pl
from jax.experimental.pallas import tpu as pltpu
from jax.experimental.pallas import tpu_sc as plsc

D = 768
H = 12
DH = 64
T = 8
NPF = 196
S = T * NPF
E = 8
K = 2
DFF = 3072
QB = 4
EPS = 1e-5

NT = 2 * S
BLK = 256
NB = 20
PADDED = NB * BLK
CW = 112


def _embed_body(p_ref, w_ref, b_ref, pos_ref, temp_ref, o_ref):
    mm = jnp.dot(p_ref[...].astype(jnp.bfloat16), w_ref[...].astype(jnp.bfloat16),
                 preferred_element_type=jnp.float32)
    add = pos_ref[...][None, :, :] + temp_ref[...][:, None, :]
    o_ref[...] = mm + b_ref[...][None, :] + add.reshape(S, D)


def _embed(patches, w, b, pos, temp):
    return pl.pallas_call(
        _embed_body,
        out_shape=jax.ShapeDtypeStruct((S, D), jnp.float32),
    )(patches, w, b, pos, temp)


def _ln_body(x_ref, g_ref, b_ref, o_ref):
    x = x_ref[...]
    m = jnp.mean(x, axis=-1, keepdims=True)
    v = jnp.mean((x - m) ** 2, axis=-1, keepdims=True)
    o_ref[...] = (x - m) * jax.lax.rsqrt(v + EPS) * g_ref[...][None, :] + b_ref[...][None, :]


def _layernorm(x, g, b):
    return pl.pallas_call(
        _ln_body,
        out_shape=jax.ShapeDtypeStruct(x.shape, jnp.float32),
    )(x, g, b)


def _qkv_body(h_ref, wq_ref, wk_ref, wv_ref, bq_ref, bk_ref, bv_ref,
              q_ref, k_ref, v_ref):
    h = h_ref[...].astype(jnp.bfloat16)
    wq = wq_ref[0].astype(jnp.bfloat16)
    wk = wk_ref[0].astype(jnp.bfloat16)
    wv = wv_ref[0].astype(jnp.bfloat16)
    q_ref[0] = jnp.dot(h, wq, preferred_element_type=jnp.float32) + bq_ref[0]
    k_ref[0] = jnp.dot(h, wk, preferred_element_type=jnp.float32) + bk_ref[0]
    v_ref[0] = jnp.dot(h, wv, preferred_element_type=jnp.float32) + bv_ref[0]


def _qkv(hln, wq, wk, wv, bq, bk, bv):
    spec_w = pl.BlockSpec((1, D, DH), lambda h: (h, 0, 0))
    spec_b = pl.BlockSpec((1, 1, DH), lambda h: (h, 0, 0))
    spec_o = pl.BlockSpec((1, S, DH), lambda h: (h, 0, 0))
    return pl.pallas_call(
        _qkv_body,
        grid=(H,),
        in_specs=[pl.BlockSpec((S, D), lambda h: (0, 0)),
                  spec_w, spec_w, spec_w, spec_b, spec_b, spec_b],
        out_specs=[spec_o, spec_o, spec_o],
        out_shape=[jax.ShapeDtypeStruct((H, S, DH), jnp.float32)] * 3,
    )(hln, wq, wk, wv, bq, bk, bv)


def _attn_body(q_ref, k_ref, v_ref, tb_ref, o_ref):
    qb = pl.program_id(1)
    q = q_ref[0]
    k = k_ref[0]
    v = v_ref[0]
    tb = tb_ref[0]
    logits = jax.lax.dot_general(
        q.astype(jnp.bfloat16), k.astype(jnp.bfloat16), (((1,), (1,)), ((), ())),
        preferred_element_type=jnp.float32) * (1.0 / 8.0)
    fr_iota = jax.lax.broadcasted_iota(jnp.int32, (T, T), 0)
    selA = (fr_iota == 2 * qb).astype(jnp.float32)
    selB = (fr_iota == 2 * qb + 1).astype(jnp.float32)
    tbA = jnp.sum(tb * selA, axis=0, keepdims=True)
    tbB = jnp.sum(tb * selB, axis=0, keepdims=True)
    colf = jax.lax.broadcasted_iota(jnp.int32, (1, S), 1) // NPF
    rowA = jnp.zeros((1, S), jnp.float32)
    rowB = jnp.zeros((1, S), jnp.float32)
    for f in range(T):
        m = (colf == f).astype(jnp.float32)
        rowA = rowA + m * tbA[:, f:f + 1]
        rowB = rowB + m * tbB[:, f:f + 1]
    rin = jax.lax.broadcasted_iota(jnp.int32, (2 * NPF, 1), 0)
    bias = jnp.where(rin < NPF, rowA, rowB)
    logits = logits + bias
    m = jnp.max(logits, axis=-1, keepdims=True)
    p = jnp.exp(logits - m)
    a = p / jnp.sum(p, axis=-1, keepdims=True)
    o_ref[0] = jnp.dot(a.astype(jnp.bfloat16), v.astype(jnp.bfloat16),
                       preferred_element_type=jnp.float32)


def _attention(q, k, v, tbias):
    SQ = S // QB
    return pl.pallas_call(
        _attn_body,
        grid=(H, QB),
        in_specs=[pl.BlockSpec((1, SQ, DH), lambda h, qb: (h, qb, 0)),
                  pl.BlockSpec((1, S, DH), lambda h, qb: (h, 0, 0)),
                  pl.BlockSpec((1, S, DH), lambda h, qb: (h, 0, 0)),
                  pl.BlockSpec((1, T, T), lambda h, qb: (h, 0, 0))],
        out_specs=pl.BlockSpec((1, SQ, DH), lambda h, qb: (h, qb, 0)),
        out_shape=jax.ShapeDtypeStruct((H, S, DH), jnp.float32),
    )(q, k, v, tbias)


def _proj_body(o_ref, wo_ref, bo_ref, x_ref, y_ref):
    h = pl.program_id(0)

    @pl.when(h == 0)
    def _():
        y_ref[...] = x_ref[...] + bo_ref[...][None, :]

    y_ref[...] += jnp.dot(o_ref[0].astype(jnp.bfloat16),
                          wo_ref[0].astype(jnp.bfloat16),
                          preferred_element_type=jnp.float32)


def _proj_residual(o_heads, wo, bo, x):
    return pl.pallas_call(
        _proj_body,
        grid=(H,),
        in_specs=[pl.BlockSpec((1, S, DH), lambda h: (h, 0, 0)),
                  pl.BlockSpec((1, DH, D), lambda h: (h, 0, 0)),
                  pl.BlockSpec((D,), lambda h: (0,)),
                  pl.BlockSpec((S, D), lambda h: (0, 0))],
        out_specs=pl.BlockSpec((S, D), lambda h: (0, 0)),
        out_shape=jax.ShapeDtypeStruct((S, D), jnp.float32),
    )(o_heads, wo, bo, x)


def _router_body(x_ref, g_ref, b_ref, wr_ref, br_ref, ts_ref, wt_ref, h_ref, rl_ref):
    x = x_ref[...]
    m = jnp.mean(x, axis=-1, keepdims=True)
    v = jnp.mean((x - m) ** 2, axis=-1, keepdims=True)
    h = (x - m) * jax.lax.rsqrt(v + EPS) * g_ref[...][None, :] + b_ref[...][None, :]
    h_ref[...] = h
    tbias = jnp.dot(ts_ref[...].astype(jnp.bfloat16),
                    wt_ref[...].astype(jnp.bfloat16),
                    preferred_element_type=jnp.float32)
    rl_ref[...] = (jnp.dot(h.astype(jnp.bfloat16), wr_ref[...].astype(jnp.bfloat16),
                           preferred_element_type=jnp.float32)
                   + br_ref[...][None, :] + tbias)


def _router(x, g, b, wr, br, text_state, wt):
    return pl.pallas_call(
        _router_body,
        out_shape=[jax.ShapeDtypeStruct((S, D), jnp.float32),
                   jax.ShapeDtypeStruct((S, E), jnp.float32)],
    )(x, g, b, wr, br, text_state, wt)


def _route_body(rl_ref, gd_ref, loads_ref, pos_ref, ebact_ref):
    r = rl_ref[...]
    iota = jax.lax.broadcasted_iota(jnp.int32, (S, E), 1)
    m1 = jnp.max(r, axis=1, keepdims=True)
    i1 = jnp.min(jnp.where(r == m1, iota, E), axis=1, keepdims=True)
    mask1b = iota == i1
    r2 = jnp.where(mask1b, -jnp.inf, r)
    m2 = jnp.max(r2, axis=1, keepdims=True)
    i2 = jnp.min(jnp.where(r2 == m2, iota, E), axis=1, keepdims=True)
    mask2b = iota == i2
    d = jnp.exp(m2 - m1)
    g1 = 1.0 / (1.0 + d)
    g2 = d / (1.0 + d)
    gd = jnp.where(mask1b, g1, 0.0) + jnp.where(mask2b, g2, 0.0)
    gd_ref[...] = gd
    loads_ref[...] = jnp.sum(gd, axis=0, keepdims=True) * (1.0 / S)

    mask1 = mask1b.astype(jnp.float32)
    mask2 = mask2b.astype(jnp.float32)
    ri = jax.lax.broadcasted_iota(jnp.int32, (S, S), 0)
    ci = jax.lax.broadcasted_iota(jnp.int32, (S, S), 1)
    tri = (ri >= ci).astype(jnp.float32)
    m12 = jnp.concatenate([mask1, mask2], axis=1)
    c12 = jnp.dot(tri, m12, preferred_element_type=jnp.float32,
                  precision=jax.lax.Precision.HIGHEST)
    c1 = c12[:, :E]
    c2 = c12[:, E:]
    cnt1 = c1[S - 1:S, :]
    cnt = cnt1 + c2[S - 1:S, :]
    nb = jnp.floor((cnt + float(BLK - 1)) * (1.0 / BLK))
    ei = jax.lax.broadcasted_iota(jnp.int32, (E, E), 0)
    ej = jax.lax.broadcasted_iota(jnp.int32, (E, E), 1)
    triE = (ei <= ej).astype(jnp.float32)
    cumnb = jnp.dot(nb, triE, preferred_element_type=jnp.float32,
                    precision=jax.lax.Precision.HIGHEST)
    seg = (cumnb - nb) * float(BLK)
    rank1 = c1 - mask1
    rank2 = cnt1 + c2 - mask2
    pos1 = jnp.sum(mask1 * (seg + rank1), axis=1, keepdims=True)
    pos2 = jnp.sum(mask2 * (seg + rank2), axis=1, keepdims=True)
    pos_ref[...] = jnp.concatenate([pos1, pos2], axis=1).astype(jnp.int32)

    bif = jax.lax.broadcasted_iota(jnp.int32, (2 * E * K, 1), 0).astype(jnp.float32)
    ebcol = jnp.sum((bif >= cumnb).astype(jnp.float32), axis=1, keepdims=True)
    ebcol = jnp.minimum(ebcol, float(E - 1))
    actcol = (bif < cumnb[:, E - 1:E]).astype(jnp.float32)
    ebact_ref[...] = jnp.concatenate([ebcol, actcol], axis=1).astype(jnp.int32)


def _route_tc(rl):
    return pl.pallas_call(
        _route_body,
        out_shape=[jax.ShapeDtypeStruct((S, E), jnp.float32),
                   jax.ShapeDtypeStruct((1, E), jnp.float32),
                   jax.ShapeDtypeStruct((S, 2), jnp.int32),
                   jax.ShapeDtypeStruct((32, 2), jnp.int32)],
    )(rl)


_SC_MESH = dict(core_axis_name="c", subcore_axis_name="s")


def _sc_scatter_body(h2_hbm, pos_hbm, out_hbm, idxv, rows, sem):
    c = lax.axis_index("c")
    s = lax.axis_index("s")
    w = s * 2 + c

    @pl.when(w < 28)
    def _():
        abase = w * CW
        tbase = abase - jnp.where(abase >= S, S, 0)
        pltpu.sync_copy(pos_hbm.at[pl.ds(abase, CW)], idxv)
        pltpu.sync_copy(h2_hbm.at[pl.ds(tbase, CW)], rows)
        pltpu.async_copy(rows, out_hbm.at[idxv], sem).wait()


def _sc_scatter(h2, pos_flat):
    f = functools.partial(
        pl.kernel,
        out_type=jax.ShapeDtypeStruct((PADDED, D), jnp.float32),
        mesh=plsc.VectorSubcoreMesh(**_SC_MESH),
        scratch_types=[
            pltpu.VMEM((CW,), jnp.int32),
            pltpu.VMEM((CW, D), jnp.float32),
            pltpu.SemaphoreType.DMA,
        ])(_sc_scatter_body)
    return f(h2, pos_flat)


def _sc_comb_body(cs_hbm, pos_hbm, out_hbm, idxv, rows, sem):
    c = lax.axis_index("c")
    s = lax.axis_index("s")
    w = s * 2 + c

    @pl.when(w < 28)
    def _():
        abase = w * CW
        pltpu.sync_copy(pos_hbm.at[pl.ds(abase, CW)], idxv)
        pltpu.async_copy(cs_hbm.at[idxv], rows, sem).wait()
        pltpu.sync_copy(rows, out_hbm.at[pl.ds(abase, CW)])


def _sc_comb(cs, pos_flat):
    f = functools.partial(
        pl.kernel,
        out_type=jax.ShapeDtypeStruct((NT, D), jnp.float32),
        mesh=plsc.VectorSubcoreMesh(**_SC_MESH),
        scratch_types=[
            pltpu.VMEM((CW,), jnp.int32),
            pltpu.VMEM((CW, D), jnp.float32),
            pltpu.SemaphoreType.DMA,
        ])(_sc_comb_body)
    return f(cs, pos_flat)


def _gmm_body(eb_ref, act_ref, rows_ref, w1_ref, bb1_ref, w2_ref, bb2_ref, o_ref):
    b = pl.program_id(0)

    @pl.when(act_ref[b] == 1)
    def _():
        hh = jax.nn.gelu(
            jnp.dot(rows_ref[...].astype(jnp.bfloat16),
                    w1_ref[0].astype(jnp.bfloat16),
                    preferred_element_type=jnp.float32) + bb1_ref[0])
        o_ref[...] = (jnp.dot(hh.astype(jnp.bfloat16),
                              w2_ref[0].astype(jnp.bfloat16),
                              preferred_element_type=jnp.float32)
                      + bb2_ref[0])


def _gmm(rows_sorted, w1, bb1, w2, bb2, eb, act):
    grid_spec = pltpu.PrefetchScalarGridSpec(
        num_scalar_prefetch=2,
        grid=(NB,),
        in_specs=[
            pl.BlockSpec((BLK, D), lambda b, eb, act: (b, 0)),
            pl.BlockSpec((1, D, DFF), lambda b, eb, act: (eb[b], 0, 0)),
            pl.BlockSpec((1, 1, DFF), lambda b, eb, act: (eb[b], 0, 0)),
            pl.BlockSpec((1, DFF, D), lambda b, eb, act: (eb[b], 0, 0)),
            pl.BlockSpec((1, 1, D), lambda b, eb, act: (eb[b], 0, 0)),
        ],
        out_specs=pl.BlockSpec((BLK, D), lambda b, eb, act: (b, 0)),
    )
    return pl.pallas_call(
        _gmm_body,
        grid_spec=grid_spec,
        out_shape=jax.ShapeDtypeStruct((PADDED, D), jnp.float32),
    )(eb, act, rows_sorted, w1, bb1, w2, bb2)


def _combine_body(x_ref, gd_ref, c1_ref, c2_ref, o_ref):
    g1 = jnp.max(gd_ref[...], axis=1, keepdims=True)
    g2 = 1.0 - g1
    o_ref[...] = x_ref[...] + g1 * c1_ref[...] + g2 * c2_ref[...]


def _combine(x, gd, c12):
    return pl.pallas_call(
        _combine_body,
        grid=(1,),
        in_specs=[pl.BlockSpec((S, D), lambda i: (0, 0)),
                  pl.BlockSpec((S, E), lambda i: (0, 0)),
                  pl.BlockSpec((S, D), lambda i: (0, 0)),
                  pl.BlockSpec((S, D), lambda i: (1, 0))],
        out_specs=pl.BlockSpec((S, D), lambda i: (0, 0)),
        out_shape=jax.ShapeDtypeStruct((S, D), jnp.float32),
    )(x, gd, c12, c12)


@jax.jit
def _run(video, text_state, params):
    P = 16
    B_, T_, C_, Hh, Ww = video.shape
    nps = Hh // P
    patches = video.reshape(B_, T_, C_, nps, P, nps, P)
    patches = patches.transpose(0, 1, 3, 5, 2, 4, 6).reshape(S, C_ * P * P)

    x = _embed(patches, params['W_patch'], params['b_patch'],
               params['pos'], params['temp'])

    loads = []
    for lp in params['layers']:
        hln = _layernorm(x, lp['g1'], lp['b1'])
        wqkv = lp['Wqkv'].reshape(D, 3, H, DH)
        wq = wqkv[:, 0].transpose(1, 0, 2)
        wk = wqkv[:, 1].transpose(1, 0, 2)
        wv = wqkv[:, 2].transpose(1, 0, 2)
        bqkv = lp['bqkv'].reshape(3, H, 1, DH)
        q, k, v = _qkv(hln, wq, wk, wv, bqkv[0], bqkv[1], bqkv[2])
        o_heads = _attention(q, k, v, lp['tbias'])
        wo = lp['Wo'].reshape(H, DH, D)
        x = _proj_residual(o_heads, wo, lp['bo'], x)

        h2, rl = _router(x, lp['g2'], lp['b2'], lp['Wr'], lp['br'],
                         text_state, lp['Wt'])
        gd, ld, posT, ebact = _route_tc(rl)
        pos_flat = posT.T.reshape(NT)
        eb = ebact[:, 0]
        act = ebact[:, 1]
        rows_sorted = _sc_scatter(h2, pos_flat)
        out_sorted = _gmm(rows_sorted, lp['W1'], lp['bb1'].reshape(E, 1, DFF),
                          lp['W2'], lp['bb2'].reshape(E, 1, D), eb, act)
        c12 = _sc_comb(out_sorted, pos_flat)
        x = _combine(x, gd, c12)
        loads.append(ld[0])

    x = _layernorm(x, params['g_f'], params['b_f'])
    return x.reshape(B_, S, D), jnp.stack(loads)


def kernel(video, text_state, params):
    return _run(video, text_state, params)


# speedup vs baseline: 1.1519x; 1.0295x over previous
import functools

import jax
import jax.numpy as jnp
from jax import lax
from jax.experimental import pallas as pl
from jax.experimental.pallas import tpu as pltpu
from jax.experimental.pallas import tpu_sc as plsc

D = 768
H = 12
DH = 64
T = 8
NPF = 196
S = T * NPF
E = 8
K = 2
DFF = 3072
QB = 4
EPS = 1e-5

NT = 2 * S
BLK = 256
NB = 20
PADDED = NB * BLK
CW = 112


def _lnorm(x, g, b):
    m = jnp.mean(x, axis=-1, keepdims=True)
    v = jnp.mean((x - m) ** 2, axis=-1, keepdims=True)
    return (x - m) * jax.lax.rsqrt(v + EPS) * g[None, :] + b[None, :]


def _embed_body(p_ref, w_ref, b_ref, pos_ref, temp_ref, g_ref, gb_ref,
                o_ref, ln_ref):
    mm = jnp.dot(p_ref[...].astype(jnp.bfloat16), w_ref[...].astype(jnp.bfloat16),
                 preferred_element_type=jnp.float32)
    add = pos_ref[...][None, :, :] + temp_ref[...][:, None, :]
    x = mm + b_ref[...][None, :] + add.reshape(S, D)
    o_ref[...] = x
    ln_ref[...] = _lnorm(x, g_ref[...], gb_ref[...])


def _embed_ln(patches, w, b, pos, temp, g, gb):
    return pl.pallas_call(
        _embed_body,
        out_shape=[jax.ShapeDtypeStruct((S, D), jnp.float32)] * 2,
    )(patches, w, b, pos, temp, g, gb)


def _qkv_body(h_ref, wq_ref, wk_ref, wv_ref, bq_ref, bk_ref, bv_ref,
              q_ref, k_ref, v_ref):
    h = h_ref[...].astype(jnp.bfloat16)
    wq = wq_ref[0].astype(jnp.bfloat16)
    wk = wk_ref[0].astype(jnp.bfloat16)
    wv = wv_ref[0].astype(jnp.bfloat16)
    q_ref[0] = jnp.dot(h, wq, preferred_element_type=jnp.float32) + bq_ref[0]
    k_ref[0] = jnp.dot(h, wk, preferred_element_type=jnp.float32) + bk_ref[0]
    v_ref[0] = jnp.dot(h, wv, preferred_element_type=jnp.float32) + bv_ref[0]


def _qkv(hln, wq, wk, wv, bq, bk, bv):
    spec_w = pl.BlockSpec((1, D, DH), lambda h: (h, 0, 0))
    spec_b = pl.BlockSpec((1, 1, DH), lambda h: (h, 0, 0))
    spec_o = pl.BlockSpec((1, S, DH), lambda h: (h, 0, 0))
    return pl.pallas_call(
        _qkv_body,
        grid=(H,),
        in_specs=[pl.BlockSpec((S, D), lambda h: (0, 0)),
                  spec_w, spec_w, spec_w, spec_b, spec_b, spec_b],
        out_specs=[spec_o, spec_o, spec_o],
        out_shape=[jax.ShapeDtypeStruct((H, S, DH), jnp.float32)] * 3,
    )(hln, wq, wk, wv, bq, bk, bv)


def _attn_body(q_ref, k_ref, v_ref, tb_ref, o_ref):
    qb = pl.program_id(1)
    q = q_ref[0]
    k = k_ref[0]
    v = v_ref[0]
    tb = tb_ref[0]
    logits = jax.lax.dot_general(
        q.astype(jnp.bfloat16), k.astype(jnp.bfloat16), (((1,), (1,)), ((), ())),
        preferred_element_type=jnp.float32) * (1.0 / 8.0)
    fr_iota = jax.lax.broadcasted_iota(jnp.int32, (T, T), 0)
    selA = (fr_iota == 2 * qb).astype(jnp.float32)
    selB = (fr_iota == 2 * qb + 1).astype(jnp.float32)
    tbA = jnp.sum(tb * selA, axis=0, keepdims=True)
    tbB = jnp.sum(tb * selB, axis=0, keepdims=True)
    colf = jax.lax.broadcasted_iota(jnp.int32, (1, S), 1) // NPF
    rowA = jnp.zeros((1, S), jnp.float32)
    rowB = jnp.zeros((1, S), jnp.float32)
    for f in range(T):
        m = (colf == f).astype(jnp.float32)
        rowA = rowA + m * tbA[:, f:f + 1]
        rowB = rowB + m * tbB[:, f:f + 1]
    rin = jax.lax.broadcasted_iota(jnp.int32, (2 * NPF, 1), 0)
    bias = jnp.where(rin < NPF, rowA, rowB)
    logits = logits + bias
    m = jnp.max(logits, axis=-1, keepdims=True)
    p = jnp.exp(logits - m)
    a = p / jnp.sum(p, axis=-1, keepdims=True)
    o_ref[0] = jnp.dot(a.astype(jnp.bfloat16), v.astype(jnp.bfloat16),
                       preferred_element_type=jnp.float32)


def _attention(q, k, v, tbias):
    SQ = S // QB
    return pl.pallas_call(
        _attn_body,
        grid=(H, QB),
        in_specs=[pl.BlockSpec((1, SQ, DH), lambda h, qb: (h, qb, 0)),
                  pl.BlockSpec((1, S, DH), lambda h, qb: (h, 0, 0)),
                  pl.BlockSpec((1, S, DH), lambda h, qb: (h, 0, 0)),
                  pl.BlockSpec((1, T, T), lambda h, qb: (h, 0, 0))],
        out_specs=pl.BlockSpec((1, SQ, DH), lambda h, qb: (h, qb, 0)),
        out_shape=jax.ShapeDtypeStruct((H, S, DH), jnp.float32),
    )(q, k, v, tbias)


def _proj_body(o_ref, wo_ref, bo_ref, x_ref, g_ref, b_ref, wr_ref, br_ref,
               ts_ref, wt_ref, y_ref, h_ref, rl_ref):
    h = pl.program_id(0)

    @pl.when(h == 0)
    def _():
        y_ref[...] = x_ref[...] + bo_ref[...][None, :]

    y_ref[...] += jnp.dot(o_ref[0].astype(jnp.bfloat16),
                          wo_ref[0].astype(jnp.bfloat16),
                          preferred_element_type=jnp.float32)

    @pl.when(h == H - 1)
    def _():
        hn = _lnorm(y_ref[...], g_ref[...], b_ref[...])
        h_ref[...] = hn
        tbias = jnp.dot(ts_ref[...].astype(jnp.bfloat16),
                        wt_ref[...].astype(jnp.bfloat16),
                        preferred_element_type=jnp.float32)
        rl_ref[...] = (jnp.dot(hn.astype(jnp.bfloat16),
                               wr_ref[...].astype(jnp.bfloat16),
                               preferred_element_type=jnp.float32)
                       + br_ref[...][None, :] + tbias)


def _proj_router(o_heads, wo, bo, x, g, b, wr, br, text_state, wt):
    cst = lambda h: (0, 0)
    return pl.pallas_call(
        _proj_body,
        grid=(H,),
        in_specs=[pl.BlockSpec((1, S, DH), lambda h: (h, 0, 0)),
                  pl.BlockSpec((1, DH, D), lambda h: (h, 0, 0)),
                  pl.BlockSpec((D,), lambda h: (0,)),
                  pl.BlockSpec((S, D), cst),
                  pl.BlockSpec((D,), lambda h: (0,)),
                  pl.BlockSpec((D,), lambda h: (0,)),
                  pl.BlockSpec((D, E), cst),
                  pl.BlockSpec((E,), lambda h: (0,)),
                  pl.BlockSpec(text_state.shape, cst),
                  pl.BlockSpec(wt.shape, cst)],
        out_specs=[pl.BlockSpec((S, D), cst),
                   pl.BlockSpec((S, D), cst),
                   pl.BlockSpec((S, E), cst)],
        out_shape=[jax.ShapeDtypeStruct((S, D), jnp.float32),
                   jax.ShapeDtypeStruct((S, D), jnp.float32),
                   jax.ShapeDtypeStruct((S, E), jnp.float32)],
    )(o_heads, wo, bo, x, g, b, wr, br, text_state, wt)


def _route_body(rl_ref, gd_ref, loads_ref, pos_ref, ebact_ref):
    r = rl_ref[...]
    iota = jax.lax.broadcasted_iota(jnp.int32, (S, E), 1)
    m1 = jnp.max(r, axis=1, keepdims=True)
    i1 = jnp.min(jnp.where(r == m1, iota, E), axis=1, keepdims=True)
    mask1b = iota == i1
    r2 = jnp.where(mask1b, -jnp.inf, r)
    m2 = jnp.max(r2, axis=1, keepdims=True)
    i2 = jnp.min(jnp.where(r2 == m2, iota, E), axis=1, keepdims=True)
    mask2b = iota == i2
    d = jnp.exp(m2 - m1)
    g1 = 1.0 / (1.0 + d)
    g2 = d / (1.0 + d)
    gd = jnp.where(mask1b, g1, 0.0) + jnp.where(mask2b, g2, 0.0)
    gd_ref[...] = gd
    loads_ref[...] = jnp.sum(gd, axis=0, keepdims=True) * (1.0 / S)

    mask1 = mask1b.astype(jnp.float32)
    mask2 = mask2b.astype(jnp.float32)
    ri = jax.lax.broadcasted_iota(jnp.int32, (S, S), 0)
    ci = jax.lax.broadcasted_iota(jnp.int32, (S, S), 1)
    tri = (ri >= ci).astype(jnp.float32)
    m12 = jnp.concatenate([mask1, mask2], axis=1)
    c12 = jnp.dot(tri, m12, preferred_element_type=jnp.float32,
                  precision=jax.lax.Precision.HIGHEST)
    c1 = c12[:, :E]
    c2 = c12[:, E:]
    cnt1 = c1[S - 1:S, :]
    cnt = cnt1 + c2[S - 1:S, :]
    nb = jnp.floor((cnt + float(BLK - 1)) * (1.0 / BLK))
    ei = jax.lax.broadcasted_iota(jnp.int32, (E, E), 0)
    ej = jax.lax.broadcasted_iota(jnp.int32, (E, E), 1)
    triE = (ei <= ej).astype(jnp.float32)
    cumnb = jnp.dot(nb, triE, preferred_element_type=jnp.float32,
                    precision=jax.lax.Precision.HIGHEST)
    seg = (cumnb - nb) * float(BLK)
    rank1 = c1 - mask1
    rank2 = cnt1 + c2 - mask2
    pos1 = jnp.sum(mask1 * (seg + rank1), axis=1, keepdims=True)
    pos2 = jnp.sum(mask2 * (seg + rank2), axis=1, keepdims=True)
    pos_ref[...] = jnp.concatenate([pos1, pos2], axis=1).astype(jnp.int32)

    bif = jax.lax.broadcasted_iota(jnp.int32, (2 * E * K, 1), 0).astype(jnp.float32)
    ebcol = jnp.sum((bif >= cumnb).astype(jnp.float32), axis=1, keepdims=True)
    ebcol = jnp.minimum(ebcol, float(E - 1))
    actcol = (bif < cumnb[:, E - 1:E]).astype(jnp.float32)
    ebact_ref[...] = jnp.concatenate([ebcol, actcol], axis=1).astype(jnp.int32)


def _route_tc(rl):
    return pl.pallas_call(
        _route_body,
        out_shape=[jax.ShapeDtypeStruct((S, E), jnp.float32),
                   jax.ShapeDtypeStruct((1, E), jnp.float32),
                   jax.ShapeDtypeStruct((S, 2), jnp.int32),
                   jax.ShapeDtypeStruct((32, 2), jnp.int32)],
    )(rl)


_SC_MESH = dict(core_axis_name="c", subcore_axis_name="s")


def _sc_scatter_body(h2_hbm, pos_hbm, out_hbm, idxv, rows, sem):
    c = lax.axis_index("c")
    s = lax.axis_index("s")
    w = s * 2 + c

    @pl.when(w < 28)
    def _():
        abase = w * CW
        tbase = abase - jnp.where(abase >= S, S, 0)
        pltpu.sync_copy(pos_hbm.at[pl.ds(abase, CW)], idxv)
        pltpu.sync_copy(h2_hbm.at[pl.ds(tbase, CW)], rows)
        pltpu.async_copy(rows, out_hbm.at[idxv], sem).wait()


def _sc_scatter(h2, pos_flat):
    f = functools.partial(
        pl.kernel,
        out_type=jax.ShapeDtypeStruct((PADDED, D), jnp.float32),
        mesh=plsc.VectorSubcoreMesh(**_SC_MESH),
        scratch_types=[
            pltpu.VMEM((CW,), jnp.int32),
            pltpu.VMEM((CW, D), jnp.float32),
            pltpu.SemaphoreType.DMA,
        ])(_sc_scatter_body)
    return f(h2, pos_flat)


def _sc_comb_body(cs_hbm, pos_hbm, out_hbm, idxv, rows, sem):
    c = lax.axis_index("c")
    s = lax.axis_index("s")
    w = s * 2 + c

    @pl.when(w < 28)
    def _():
        abase = w * CW
        pltpu.sync_copy(pos_hbm.at[pl.ds(abase, CW)], idxv)
        pltpu.async_copy(cs_hbm.at[idxv], rows, sem).wait()
        pltpu.sync_copy(rows, out_hbm.at[pl.ds(abase, CW)])


def _sc_comb(cs, pos_flat):
    f = functools.partial(
        pl.kernel,
        out_type=jax.ShapeDtypeStruct((NT, D), jnp.float32),
        mesh=plsc.VectorSubcoreMesh(**_SC_MESH),
        scratch_types=[
            pltpu.VMEM((CW,), jnp.int32),
            pltpu.VMEM((CW, D), jnp.float32),
            pltpu.SemaphoreType.DMA,
        ])(_sc_comb_body)
    return f(cs, pos_flat)


def _gmm_body(eb_ref, act_ref, rows_ref, w1_ref, bb1_ref, w2_ref, bb2_ref, o_ref):
    b = pl.program_id(0)

    @pl.when(act_ref[b] == 1)
    def _():
        hh = jax.nn.gelu(
            jnp.dot(rows_ref[...].astype(jnp.bfloat16),
                    w1_ref[0].astype(jnp.bfloat16),
                    preferred_element_type=jnp.float32) + bb1_ref[0])
        o_ref[...] = (jnp.dot(hh.astype(jnp.bfloat16),
                              w2_ref[0].astype(jnp.bfloat16),
                              preferred_element_type=jnp.float32)
                      + bb2_ref[0])


def _gmm(rows_sorted, w1, bb1, w2, bb2, eb, act):
    grid_spec = pltpu.PrefetchScalarGridSpec(
        num_scalar_prefetch=2,
        grid=(NB,),
        in_specs=[
            pl.BlockSpec((BLK, D), lambda b, eb, act: (b, 0)),
            pl.BlockSpec((1, D, DFF), lambda b, eb, act: (eb[b], 0, 0)),
            pl.BlockSpec((1, 1, DFF), lambda b, eb, act: (eb[b], 0, 0)),
            pl.BlockSpec((1, DFF, D), lambda b, eb, act: (eb[b], 0, 0)),
            pl.BlockSpec((1, 1, D), lambda b, eb, act: (eb[b], 0, 0)),
        ],
        out_specs=pl.BlockSpec((BLK, D), lambda b, eb, act: (b, 0)),
    )
    return pl.pallas_call(
        _gmm_body,
        grid_spec=grid_spec,
        out_shape=jax.ShapeDtypeStruct((PADDED, D), jnp.float32),
    )(eb, act, rows_sorted, w1, bb1, w2, bb2)


def _combine_body(x_ref, gd_ref, c1_ref, c2_ref, g_ref, b_ref, o_ref, ln_ref):
    g1 = jnp.max(gd_ref[...], axis=1, keepdims=True)
    g2 = 1.0 - g1
    xn = x_ref[...] + g1 * c1_ref[...] + g2 * c2_ref[...]
    o_ref[...] = xn
    ln_ref[...] = _lnorm(xn, g_ref[...], b_ref[...])


def _combine_ln(x, gd, c12, g, b):
    return pl.pallas_call(
        _combine_body,
        grid=(1,),
        in_specs=[pl.BlockSpec((S, D), lambda i: (0, 0)),
                  pl.BlockSpec((S, E), lambda i: (0, 0)),
                  pl.BlockSpec((S, D), lambda i: (0, 0)),
                  pl.BlockSpec((S, D), lambda i: (1, 0)),
                  pl.BlockSpec((D,), lambda i: (0,)),
                  pl.BlockSpec((D,), lambda i: (0,))],
        out_specs=[pl.BlockSpec((S, D), lambda i: (0, 0)),
                   pl.BlockSpec((S, D), lambda i: (0, 0))],
        out_shape=[jax.ShapeDtypeStruct((S, D), jnp.float32)] * 2,
    )(x, gd, c12, c12, g, b)


@jax.jit
def _run(video, text_state, params):
    P = 16
    B_, T_, C_, Hh, Ww = video.shape
    nps = Hh // P
    patches = video.reshape(B_, T_, C_, nps, P, nps, P)
    patches = patches.transpose(0, 1, 3, 5, 2, 4, 6).reshape(S, C_ * P * P)

    layers = params['layers']
    x, hln = _embed_ln(patches, params['W_patch'], params['b_patch'],
                       params['pos'], params['temp'],
                       layers[0]['g1'], layers[0]['b1'])

    loads = []
    for li, lp in enumerate(layers):
        wqkv = lp['Wqkv'].reshape(D, 3, H, DH)
        wq = wqkv[:, 0].transpose(1, 0, 2)
        wk = wqkv[:, 1].transpose(1, 0, 2)
        wv = wqkv[:, 2].transpose(1, 0, 2)
        bqkv = lp['bqkv'].reshape(3, H, 1, DH)
        q, k, v = _qkv(hln, wq, wk, wv, bqkv[0], bqkv[1], bqkv[2])
        o_heads = _attention(q, k, v, lp['tbias'])
        wo = lp['Wo'].reshape(H, DH, D)
        y, h2, rl = _proj_router(o_heads, wo, lp['bo'], x,
                                 lp['g2'], lp['b2'], lp['Wr'], lp['br'],
                                 text_state, lp['Wt'])
        gd, ld, posT, ebact = _route_tc(rl)
        pos_flat = posT.T.reshape(NT)
        eb = ebact[:, 0]
        act = ebact[:, 1]
        rows_sorted = _sc_scatter(h2, pos_flat)
        out_sorted = _gmm(rows_sorted, lp['W1'], lp['bb1'].reshape(E, 1, DFF),
                          lp['W2'], lp['bb2'].reshape(E, 1, D), eb, act)
        c12 = _sc_comb(out_sorted, pos_flat)
        if li + 1 < len(layers):
            ng, nb_ = layers[li + 1]['g1'], layers[li + 1]['b1']
        else:
            ng, nb_ = params['g_f'], params['b_f']
        x, hln = _combine_ln(y, gd, c12, ng, nb_)
        loads.append(ld[0])

    return hln.reshape(B_, S, D), jnp.stack(loads)


def kernel(video, text_state, params):
    return _run(video, text_state, params)


# speedup vs baseline: 1.1656x; 1.0119x over previous
import functools

import jax
import jax.numpy as jnp
from jax import lax
from jax.experimental import pallas as pl
from jax.experimental.pallas import tpu as pltpu
from jax.experimental.pallas import tpu_sc as plsc

D = 768
H = 12
DH = 64
T = 8
NPF = 196
S = T * NPF
E = 8
K = 2
DFF = 3072
QB = 4
EPS = 1e-5

NT = 2 * S
BLK = 256
NB = 20
PADDED = NB * BLK
CW = 112


def _lnorm(x, g, b):
    m = jnp.mean(x, axis=-1, keepdims=True)
    v = jnp.mean((x - m) ** 2, axis=-1, keepdims=True)
    return (x - m) * jax.lax.rsqrt(v + EPS) * g[None, :] + b[None, :]


def _embed_body(p_ref, w_ref, b_ref, pos_ref, temp_ref, g_ref, gb_ref,
                o_ref, ln_ref):
    mm = jnp.dot(p_ref[...], w_ref[...], preferred_element_type=jnp.float32)
    add = pos_ref[...][None, :, :] + temp_ref[...][:, None, :]
    x = mm + b_ref[...][None, :] + add.reshape(S, D)
    o_ref[...] = x
    ln_ref[...] = _lnorm(x, g_ref[...], gb_ref[...])


def _embed_ln(patches, w, b, pos, temp, g, gb):
    return pl.pallas_call(
        _embed_body,
        out_shape=[jax.ShapeDtypeStruct((S, D), jnp.float32)] * 2,
    )(patches, w, b, pos, temp, g, gb)


def _qkv_body(h_ref, wq_ref, wk_ref, wv_ref, bq_ref, bk_ref, bv_ref,
              q_ref, k_ref, v_ref):
    h = h_ref[...]
    wq = wq_ref[0]
    wk = wk_ref[0]
    wv = wv_ref[0]
    q_ref[0] = jnp.dot(h, wq, preferred_element_type=jnp.float32) + bq_ref[0]
    k_ref[0] = jnp.dot(h, wk, preferred_element_type=jnp.float32) + bk_ref[0]
    v_ref[0] = jnp.dot(h, wv, preferred_element_type=jnp.float32) + bv_ref[0]


def _qkv(hln, wq, wk, wv, bq, bk, bv):
    spec_w = pl.BlockSpec((1, D, DH), lambda h: (h, 0, 0))
    spec_b = pl.BlockSpec((1, 1, DH), lambda h: (h, 0, 0))
    spec_o = pl.BlockSpec((1, S, DH), lambda h: (h, 0, 0))
    return pl.pallas_call(
        _qkv_body,
        grid=(H,),
        in_specs=[pl.BlockSpec((S, D), lambda h: (0, 0)),
                  spec_w, spec_w, spec_w, spec_b, spec_b, spec_b],
        out_specs=[spec_o, spec_o, spec_o],
        out_shape=[jax.ShapeDtypeStruct((H, S, DH), jnp.float32)] * 3,
    )(hln, wq, wk, wv, bq, bk, bv)


def _attn_body(q_ref, k_ref, v_ref, tb_ref, o_ref):
    qb = pl.program_id(1)
    q = q_ref[0]
    k = k_ref[0]
    v = v_ref[0]
    tb = tb_ref[0]
    logits = jax.lax.dot_general(
        q, k, (((1,), (1,)), ((), ())),
        preferred_element_type=jnp.float32) * (1.0 / 8.0)
    fr_iota = jax.lax.broadcasted_iota(jnp.int32, (T, T), 0)
    selA = (fr_iota == 2 * qb).astype(jnp.float32)
    selB = (fr_iota == 2 * qb + 1).astype(jnp.float32)
    tbA = jnp.sum(tb * selA, axis=0, keepdims=True)
    tbB = jnp.sum(tb * selB, axis=0, keepdims=True)
    colf = jax.lax.broadcasted_iota(jnp.int32, (1, S), 1) // NPF
    rowA = jnp.zeros((1, S), jnp.float32)
    rowB = jnp.zeros((1, S), jnp.float32)
    for f in range(T):
        m = (colf == f).astype(jnp.float32)
        rowA = rowA + m * tbA[:, f:f + 1]
        rowB = rowB + m * tbB[:, f:f + 1]
    rin = jax.lax.broadcasted_iota(jnp.int32, (2 * NPF, 1), 0)
    bias = jnp.where(rin < NPF, rowA, rowB)
    logits = logits + bias
    m = jnp.max(logits, axis=-1, keepdims=True)
    p = jnp.exp(logits - m)
    a = p / jnp.sum(p, axis=-1, keepdims=True)
    o_ref[0] = jnp.dot(a, v, preferred_element_type=jnp.float32)


def _attention(q, k, v, tbias):
    SQ = S // QB
    return pl.pallas_call(
        _attn_body,
        grid=(H, QB),
        in_specs=[pl.BlockSpec((1, SQ, DH), lambda h, qb: (h, qb, 0)),
                  pl.BlockSpec((1, S, DH), lambda h, qb: (h, 0, 0)),
                  pl.BlockSpec((1, S, DH), lambda h, qb: (h, 0, 0)),
                  pl.BlockSpec((1, T, T), lambda h, qb: (h, 0, 0))],
        out_specs=pl.BlockSpec((1, SQ, DH), lambda h, qb: (h, qb, 0)),
        out_shape=jax.ShapeDtypeStruct((H, S, DH), jnp.float32),
    )(q, k, v, tbias)


def _proj_body(o_ref, wo_ref, bo_ref, x_ref, g_ref, b_ref, wr_ref, br_ref,
               ts_ref, wt_ref, y_ref, h_ref, rl_ref):
    h = pl.program_id(0)

    @pl.when(h == 0)
    def _():
        y_ref[...] = x_ref[...] + bo_ref[...][None, :]

    y_ref[...] += jnp.dot(o_ref[0], wo_ref[0],
                          preferred_element_type=jnp.float32)

    @pl.when(h == H - 1)
    def _():
        hn = _lnorm(y_ref[...], g_ref[...], b_ref[...])
        h_ref[...] = hn
        tbias = jnp.dot(ts_ref[...], wt_ref[...],
                        preferred_element_type=jnp.float32)
        rl_ref[...] = (jnp.dot(hn, wr_ref[...],
                               preferred_element_type=jnp.float32)
                       + br_ref[...][None, :] + tbias)


def _proj_router(o_heads, wo, bo, x, g, b, wr, br, text_state, wt):
    cst = lambda h: (0, 0)
    return pl.pallas_call(
        _proj_body,
        grid=(H,),
        in_specs=[pl.BlockSpec((1, S, DH), lambda h: (h, 0, 0)),
                  pl.BlockSpec((1, DH, D), lambda h: (h, 0, 0)),
                  pl.BlockSpec((D,), lambda h: (0,)),
                  pl.BlockSpec((S, D), cst),
                  pl.BlockSpec((D,), lambda h: (0,)),
                  pl.BlockSpec((D,), lambda h: (0,)),
                  pl.BlockSpec((D, E), cst),
                  pl.BlockSpec((E,), lambda h: (0,)),
                  pl.BlockSpec(text_state.shape, cst),
                  pl.BlockSpec(wt.shape, cst)],
        out_specs=[pl.BlockSpec((S, D), cst),
                   pl.BlockSpec((S, D), cst),
                   pl.BlockSpec((S, E), cst)],
        out_shape=[jax.ShapeDtypeStruct((S, D), jnp.float32),
                   jax.ShapeDtypeStruct((S, D), jnp.float32),
                   jax.ShapeDtypeStruct((S, E), jnp.float32)],
    )(o_heads, wo, bo, x, g, b, wr, br, text_state, wt)


def _route_body(rl_ref, gd_ref, loads_ref, pos_ref, ebact_ref):
    r = rl_ref[...]
    iota = jax.lax.broadcasted_iota(jnp.int32, (S, E), 1)
    m1 = jnp.max(r, axis=1, keepdims=True)
    i1 = jnp.min(jnp.where(r == m1, iota, E), axis=1, keepdims=True)
    mask1b = iota == i1
    r2 = jnp.where(mask1b, -jnp.inf, r)
    m2 = jnp.max(r2, axis=1, keepdims=True)
    i2 = jnp.min(jnp.where(r2 == m2, iota, E), axis=1, keepdims=True)
    mask2b = iota == i2
    d = jnp.exp(m2 - m1)
    g1 = 1.0 / (1.0 + d)
    g2 = d / (1.0 + d)
    gd = jnp.where(mask1b, g1, 0.0) + jnp.where(mask2b, g2, 0.0)
    gd_ref[...] = gd
    loads_ref[...] = jnp.sum(gd, axis=0, keepdims=True) * (1.0 / S)

    mask1 = mask1b.astype(jnp.float32)
    mask2 = mask2b.astype(jnp.float32)
    ri = jax.lax.broadcasted_iota(jnp.int32, (S, S), 0)
    ci = jax.lax.broadcasted_iota(jnp.int32, (S, S), 1)
    tri = (ri >= ci).astype(jnp.float32)
    m12 = jnp.concatenate([mask1, mask2], axis=1)
    c12 = jnp.dot(tri, m12, preferred_element_type=jnp.float32,
                  precision=jax.lax.Precision.HIGHEST)
    c1 = c12[:, :E]
    c2 = c12[:, E:]
    cnt1 = c1[S - 1:S, :]
    cnt = cnt1 + c2[S - 1:S, :]
    nb = jnp.floor((cnt + float(BLK - 1)) * (1.0 / BLK))
    ei = jax.lax.broadcasted_iota(jnp.int32, (E, E), 0)
    ej = jax.lax.broadcasted_iota(jnp.int32, (E, E), 1)
    triE = (ei <= ej).astype(jnp.float32)
    cumnb = jnp.dot(nb, triE, preferred_element_type=jnp.float32,
                    precision=jax.lax.Precision.HIGHEST)
    seg = (cumnb - nb) * float(BLK)
    rank1 = c1 - mask1
    rank2 = cnt1 + c2 - mask2
    pos1 = jnp.sum(mask1 * (seg + rank1), axis=1, keepdims=True)
    pos2 = jnp.sum(mask2 * (seg + rank2), axis=1, keepdims=True)
    pos_ref[...] = jnp.concatenate([pos1, pos2], axis=1).astype(jnp.int32)

    bif = jax.lax.broadcasted_iota(jnp.int32, (2 * E * K, 1), 0).astype(jnp.float32)
    ebcol = jnp.sum((bif >= cumnb).astype(jnp.float32), axis=1, keepdims=True)
    ebcol = jnp.minimum(ebcol, float(E - 1))
    actcol = (bif < cumnb[:, E - 1:E]).astype(jnp.float32)
    ebact_ref[...] = jnp.concatenate([ebcol, actcol], axis=1).astype(jnp.int32)


def _route_tc(rl):
    return pl.pallas_call(
        _route_body,
        out_shape=[jax.ShapeDtypeStruct((S, E), jnp.float32),
                   jax.ShapeDtypeStruct((1, E), jnp.float32),
                   jax.ShapeDtypeStruct((S, 2), jnp.int32),
                   jax.ShapeDtypeStruct((32, 2), jnp.int32)],
    )(rl)


_SC_MESH = dict(core_axis_name="c", subcore_axis_name="s")


def _sc_scatter_body(h2_hbm, pos_hbm, out_hbm, idxv, rows, sem):
    c = lax.axis_index("c")
    s = lax.axis_index("s")
    w = s * 2 + c

    @pl.when(w < 28)
    def _():
        abase = w * CW
        tbase = abase - jnp.where(abase >= S, S, 0)
        pltpu.sync_copy(pos_hbm.at[pl.ds(abase, CW)], idxv)
        pltpu.sync_copy(h2_hbm.at[pl.ds(tbase, CW)], rows)
        pltpu.async_copy(rows, out_hbm.at[idxv], sem).wait()


def _sc_scatter(h2, pos_flat):
    f = functools.partial(
        pl.kernel,
        out_type=jax.ShapeDtypeStruct((PADDED, D), jnp.float32),
        mesh=plsc.VectorSubcoreMesh(**_SC_MESH),
        scratch_types=[
            pltpu.VMEM((CW,), jnp.int32),
            pltpu.VMEM((CW, D), jnp.float32),
            pltpu.SemaphoreType.DMA,
        ])(_sc_scatter_body)
    return f(h2, pos_flat)


def _sc_comb_body(cs_hbm, pos_hbm, out_hbm, idxv, rows, sem):
    c = lax.axis_index("c")
    s = lax.axis_index("s")
    w = s * 2 + c

    @pl.when(w < 28)
    def _():
        abase = w * CW
        pltpu.sync_copy(pos_hbm.at[pl.ds(abase, CW)], idxv)
        pltpu.async_copy(cs_hbm.at[idxv], rows, sem).wait()
        pltpu.sync_copy(rows, out_hbm.at[pl.ds(abase, CW)])


def _sc_comb(cs, pos_flat):
    f = functools.partial(
        pl.kernel,
        out_type=jax.ShapeDtypeStruct((NT, D), jnp.float32),
        mesh=plsc.VectorSubcoreMesh(**_SC_MESH),
        scratch_types=[
            pltpu.VMEM((CW,), jnp.int32),
            pltpu.VMEM((CW, D), jnp.float32),
            pltpu.SemaphoreType.DMA,
        ])(_sc_comb_body)
    return f(cs, pos_flat)


def _gmm_body(eb_ref, act_ref, rows_ref, w1_ref, bb1_ref, w2_ref, bb2_ref, o_ref):
    b = pl.program_id(0)

    @pl.when(act_ref[b] == 1)
    def _():
        hh = jax.nn.gelu(
            jnp.dot(rows_ref[...], w1_ref[0],
                    preferred_element_type=jnp.float32) + bb1_ref[0])
        o_ref[...] = (jnp.dot(hh, w2_ref[0],
                              preferred_element_type=jnp.float32)
                      + bb2_ref[0])


def _gmm(rows_sorted, w1, bb1, w2, bb2, eb, act):
    grid_spec = pltpu.PrefetchScalarGridSpec(
        num_scalar_prefetch=2,
        grid=(NB,),
        in_specs=[
            pl.BlockSpec((BLK, D), lambda b, eb, act: (b, 0)),
            pl.BlockSpec((1, D, DFF), lambda b, eb, act: (eb[b], 0, 0)),
            pl.BlockSpec((1, 1, DFF), lambda b, eb, act: (eb[b], 0, 0)),
            pl.BlockSpec((1, DFF, D), lambda b, eb, act: (eb[b], 0, 0)),
            pl.BlockSpec((1, 1, D), lambda b, eb, act: (eb[b], 0, 0)),
        ],
        out_specs=pl.BlockSpec((BLK, D), lambda b, eb, act: (b, 0)),
    )
    return pl.pallas_call(
        _gmm_body,
        grid_spec=grid_spec,
        out_shape=jax.ShapeDtypeStruct((PADDED, D), jnp.float32),
    )(eb, act, rows_sorted, w1, bb1, w2, bb2)


def _combine_body(x_ref, gd_ref, c1_ref, c2_ref, g_ref, b_ref, o_ref, ln_ref):
    g1 = jnp.max(gd_ref[...], axis=1, keepdims=True)
    g2 = 1.0 - g1
    xn = x_ref[...] + g1 * c1_ref[...] + g2 * c2_ref[...]
    o_ref[...] = xn
    ln_ref[...] = _lnorm(xn, g_ref[...], b_ref[...])


def _combine_ln(x, gd, c12, g, b):
    return pl.pallas_call(
        _combine_body,
        grid=(1,),
        in_specs=[pl.BlockSpec((S, D), lambda i: (0, 0)),
                  pl.BlockSpec((S, E), lambda i: (0, 0)),
                  pl.BlockSpec((S, D), lambda i: (0, 0)),
                  pl.BlockSpec((S, D), lambda i: (1, 0)),
                  pl.BlockSpec((D,), lambda i: (0,)),
                  pl.BlockSpec((D,), lambda i: (0,))],
        out_specs=[pl.BlockSpec((S, D), lambda i: (0, 0)),
                   pl.BlockSpec((S, D), lambda i: (0, 0))],
        out_shape=[jax.ShapeDtypeStruct((S, D), jnp.float32)] * 2,
    )(x, gd, c12, c12, g, b)


@jax.jit
def _run(video, text_state, params):
    P = 16
    B_, T_, C_, Hh, Ww = video.shape
    nps = Hh // P
    patches = video.reshape(B_, T_, C_, nps, P, nps, P)
    patches = patches.transpose(0, 1, 3, 5, 2, 4, 6).reshape(S, C_ * P * P)

    layers = params['layers']
    x, hln = _embed_ln(patches, params['W_patch'], params['b_patch'],
                       params['pos'], params['temp'],
                       layers[0]['g1'], layers[0]['b1'])

    loads = []
    for li, lp in enumerate(layers):
        wqkv = lp['Wqkv'].reshape(D, 3, H, DH)
        wq = wqkv[:, 0].transpose(1, 0, 2)
        wk = wqkv[:, 1].transpose(1, 0, 2)
        wv = wqkv[:, 2].transpose(1, 0, 2)
        bqkv = lp['bqkv'].reshape(3, H, 1, DH)
        q, k, v = _qkv(hln, wq, wk, wv, bqkv[0], bqkv[1], bqkv[2])
        o_heads = _attention(q, k, v, lp['tbias'])
        wo = lp['Wo'].reshape(H, DH, D)
        y, h2, rl = _proj_router(o_heads, wo, lp['bo'], x,
                                 lp['g2'], lp['b2'], lp['Wr'], lp['br'],
                                 text_state, lp['Wt'])
        gd, ld, posT, ebact = _route_tc(rl)
        pos_flat = posT.T.reshape(NT)
        eb = ebact[:, 0]
        act = ebact[:, 1]
        rows_sorted = _sc_scatter(h2, pos_flat)
        out_sorted = _gmm(rows_sorted, lp['W1'], lp['bb1'].reshape(E, 1, DFF),
                          lp['W2'], lp['bb2'].reshape(E, 1, D), eb, act)
        c12 = _sc_comb(out_sorted, pos_flat)
        if li + 1 < len(layers):
            ng, nb_ = layers[li + 1]['g1'], layers[li + 1]['b1']
        else:
            ng, nb_ = params['g_f'], params['b_f']
        x, hln = _combine_ln(y, gd, c12, ng, nb_)
        loads.append(ld[0])

    return hln.reshape(B_, S, D), jnp.stack(loads)


def kernel(video, text_state, params):
    return _run(video, text_state, params)
